# Initial kernel scaffold; baseline (speedup 1.0000x reference)
import functools

import jax
import jax.numpy as jnp
from jax import lax
from jax.experimental import pallas as pl
from jax.experimental.pallas import tpu as pltpu
from jax.experimental.pallas import tpu_sc as plsc

N = 10000
E = 320000
D = 128

NC = 2
NS = 16
NW = NC * NS

CHUNK = 128
NCHUNKS = E // CHUNK
ITERS = -(-NCHUNKS // NW)
ROWS_PER_TILE = N // NS
DEGW = 8

_mesh = plsc.VectorSubcoreMesh(core_axis_name="c", subcore_axis_name="s")


def _deg_body(col_hbm, zeros8_hbm, ones8_hbm, degp_hbm, deg_sh, cidx_v, ones_v):
    c = lax.axis_index("c")
    s = lax.axis_index("s")
    wid = s * NC + c
    pltpu.sync_copy(zeros8_hbm, deg_sh.at[pl.ds(s * ROWS_PER_TILE, ROWS_PER_TILE)])
    pltpu.sync_copy(ones8_hbm, ones_v)
    plsc.subcore_barrier()

    def body(j, _):
        chunk = wid + j * NW

        @pl.when(chunk < NCHUNKS)
        def _():
            pltpu.sync_copy(col_hbm.at[pl.ds(chunk * CHUNK, CHUNK)], cidx_v)
            pltpu.sync_copy(ones_v, deg_sh.at[cidx_v], add=True)

        return _

    lax.fori_loop(0, ITERS, body, None)
    plsc.subcore_barrier()
    pltpu.sync_copy(
        deg_sh.at[pl.ds(s * ROWS_PER_TILE, ROWS_PER_TILE)],
        degp_hbm.at[c, pl.ds(s * ROWS_PER_TILE, ROWS_PER_TILE)],
    )


_deg_call = pl.kernel(
    _deg_body,
    out_type=jax.ShapeDtypeStruct((NC, N, DEGW), jnp.float32),
    mesh=_mesh,
    scratch_types=[
        pltpu.VMEM_SHARED((N, DEGW), jnp.float32),
        pltpu.VMEM((CHUNK,), jnp.int32),
        pltpu.VMEM((CHUNK, DEGW), jnp.float32),
    ],
)


def _agg_body(row_hbm, col_hbm, xs_hbm, zerosd_hbm, aggp_hbm,
              acc_sh, ridx_v, cidx_v, rows_v, sem):
    c = lax.axis_index("c")
    s = lax.axis_index("s")
    wid = s * NC + c
    pltpu.sync_copy(zerosd_hbm, acc_sh.at[pl.ds(s * ROWS_PER_TILE, ROWS_PER_TILE)])
    plsc.subcore_barrier()

    def body(j, _):
        chunk = wid + j * NW

        @pl.when(chunk < NCHUNKS)
        def _():
            pltpu.sync_copy(row_hbm.at[pl.ds(chunk * CHUNK, CHUNK)], ridx_v)
            pltpu.sync_copy(col_hbm.at[pl.ds(chunk * CHUNK, CHUNK)], cidx_v)
            pltpu.async_copy(xs_hbm.at[ridx_v], rows_v, sem).wait()
            pltpu.sync_copy(rows_v, acc_sh.at[cidx_v], add=True)

        return _

    lax.fori_loop(0, ITERS, body, None)
    plsc.subcore_barrier()
    pltpu.sync_copy(
        acc_sh.at[pl.ds(s * ROWS_PER_TILE, ROWS_PER_TILE)],
        aggp_hbm.at[c, pl.ds(s * ROWS_PER_TILE, ROWS_PER_TILE)],
    )


_agg_call = pl.kernel(
    _agg_body,
    out_type=jax.ShapeDtypeStruct((NC, N, D), jnp.float32),
    mesh=_mesh,
    scratch_types=[
        pltpu.VMEM_SHARED((N, D), jnp.float32),
        pltpu.VMEM((CHUNK,), jnp.int32),
        pltpu.VMEM((CHUNK,), jnp.int32),
        pltpu.VMEM((CHUNK, D), jnp.float32),
        pltpu.SemaphoreType.DMA,
    ],
)

_BN = 1000


def _prescale_body(degp_ref, x_ref, xs_ref):
    deg = degp_ref[0, :, 0:1] + degp_ref[1, :, 0:1]
    dis = jnp.where(deg > 0.0, lax.rsqrt(deg), 0.0)
    xs_ref[...] = x_ref[...] * dis


def _prescale(degp, x):
    return pl.pallas_call(
        _prescale_body,
        grid=(N // _BN,),
        in_specs=[
            pl.BlockSpec((NC, _BN, DEGW), lambda i: (0, i, 0)),
            pl.BlockSpec((_BN, D), lambda i: (i, 0)),
        ],
        out_specs=pl.BlockSpec((_BN, D), lambda i: (i, 0)),
        out_shape=jax.ShapeDtypeStruct((N, D), jnp.float32),
    )(degp, x)


def _final_body(degp_ref, aggp_ref, w_ref, b_ref, o_ref):
    deg = degp_ref[0, :, 0:1] + degp_ref[1, :, 0:1]
    dis = jnp.where(deg > 0.0, lax.rsqrt(deg), 0.0)
    acc = (aggp_ref[0] + aggp_ref[1]) * dis
    o_ref[...] = lax.dot_general(
        acc, w_ref[...], (((1,), (1,)), ((), ())),
        preferred_element_type=jnp.float32,
    ) + b_ref[...]


def _final(degp, aggp, W_w, W_b2):
    return pl.pallas_call(
        _final_body,
        grid=(N // _BN,),
        in_specs=[
            pl.BlockSpec((NC, _BN, DEGW), lambda i: (0, i, 0)),
            pl.BlockSpec((NC, _BN, D), lambda i: (0, i, 0)),
            pl.BlockSpec((D, D), lambda i: (0, 0)),
            pl.BlockSpec((1, D), lambda i: (0, 0)),
        ],
        out_specs=pl.BlockSpec((_BN, D), lambda i: (i, 0)),
        out_shape=jax.ShapeDtypeStruct((N, D), jnp.float32),
    )(degp, aggp, W_w, W_b2)


@jax.jit
def kernel(x, edge_index, x0, W_w, W_b):
    del x0
    row = edge_index[0]
    col = edge_index[1]
    zeros8 = jnp.zeros((ROWS_PER_TILE, DEGW), jnp.float32)
    ones8 = jnp.ones((CHUNK, DEGW), jnp.float32)
    zerosd = jnp.zeros((ROWS_PER_TILE, D), jnp.float32)
    degp = _deg_call(col, zeros8, ones8)
    xs = _prescale(degp, x)
    aggp = _agg_call(row, col, xs, zerosd)
    return _final(degp, aggp, W_w, W_b.reshape(1, D))


# speedup vs baseline: 18.4030x; 18.4030x over previous
import functools

import jax
import jax.numpy as jnp
from jax import lax
from jax.experimental import pallas as pl
from jax.experimental.pallas import tpu as pltpu
from jax.experimental.pallas import tpu_sc as plsc

N = 10000
NPAD = 10240
E = 320000
D = 128

NC = 2
NS = 16
NW = NC * NS

CHUNK = 128
NCHUNKS = E // CHUNK
ITERS = -(-NCHUNKS // NW)
ROWS_PER_TILE = NPAD // NS
DEGW = 16

_mesh = plsc.VectorSubcoreMesh(core_axis_name="c", subcore_axis_name="s")


def _deg_body(col_hbm, zeros8_hbm, ones8_hbm, degp_hbm, deg_sh, cidx_v, ones_v):
    c = lax.axis_index("c")
    s = lax.axis_index("s")
    wid = s * NC + c
    pltpu.sync_copy(zeros8_hbm, deg_sh.at[pl.ds(s * ROWS_PER_TILE, ROWS_PER_TILE)])
    pltpu.sync_copy(ones8_hbm, ones_v)
    plsc.subcore_barrier()

    def body(j, _):
        chunk = wid + j * NW

        @pl.when(chunk < NCHUNKS)
        def _():
            pltpu.sync_copy(col_hbm.at[pl.ds(chunk * CHUNK, CHUNK)], cidx_v)
            pltpu.sync_copy(ones_v, deg_sh.at[cidx_v], add=True)

        return _

    lax.fori_loop(0, ITERS, body, None)
    plsc.subcore_barrier()
    pltpu.sync_copy(
        deg_sh.at[pl.ds(s * ROWS_PER_TILE, ROWS_PER_TILE)],
        degp_hbm.at[c, pl.ds(s * ROWS_PER_TILE, ROWS_PER_TILE)],
    )


_deg_call = pl.kernel(
    _deg_body,
    out_type=jax.ShapeDtypeStruct((NC, NPAD, DEGW), jnp.float32),
    mesh=_mesh,
    scratch_types=[
        pltpu.VMEM_SHARED((NPAD, DEGW), jnp.float32),
        pltpu.VMEM((CHUNK,), jnp.int32),
        pltpu.VMEM((CHUNK, DEGW), jnp.float32),
    ],
    compiler_params=pltpu.CompilerParams(use_tc_tiling_on_sc=False),
)


def _agg_body(row_hbm, col_hbm, xs_hbm, zerosd_hbm, aggp_hbm,
              acc_sh, ridx_v, cidx_v, rows_v, sem):
    c = lax.axis_index("c")
    s = lax.axis_index("s")
    wid = s * NC + c
    pltpu.sync_copy(zerosd_hbm, acc_sh.at[pl.ds(s * ROWS_PER_TILE, ROWS_PER_TILE)])
    plsc.subcore_barrier()

    def body(j, _):
        chunk = wid + j * NW

        @pl.when(chunk < NCHUNKS)
        def _():
            pltpu.sync_copy(row_hbm.at[pl.ds(chunk * CHUNK, CHUNK)], ridx_v)
            pltpu.sync_copy(col_hbm.at[pl.ds(chunk * CHUNK, CHUNK)], cidx_v)
            pltpu.async_copy(xs_hbm.at[ridx_v], rows_v, sem).wait()
            pltpu.sync_copy(rows_v, acc_sh.at[cidx_v], add=True)

        return _

    lax.fori_loop(0, ITERS, body, None)
    plsc.subcore_barrier()
    pltpu.sync_copy(
        acc_sh.at[pl.ds(s * ROWS_PER_TILE, ROWS_PER_TILE)],
        aggp_hbm.at[c, pl.ds(s * ROWS_PER_TILE, ROWS_PER_TILE)],
    )


_agg_call = pl.kernel(
    _agg_body,
    out_type=jax.ShapeDtypeStruct((NC, NPAD, D), jnp.float32),
    mesh=_mesh,
    scratch_types=[
        pltpu.VMEM_SHARED((NPAD, D), jnp.float32),
        pltpu.VMEM((CHUNK,), jnp.int32),
        pltpu.VMEM((CHUNK,), jnp.int32),
        pltpu.VMEM((CHUNK, D), jnp.float32),
        pltpu.SemaphoreType.DMA,
    ],
)

_BN = 1024


def _prescale_body(degp_ref, x_ref, xs_ref):
    deg = degp_ref[0, :, 0:1] + degp_ref[1, :, 0:1]
    dis = jnp.where(deg > 0.0, lax.rsqrt(deg), 0.0)
    xs_ref[...] = x_ref[...] * dis


def _prescale(degp, x):
    return pl.pallas_call(
        _prescale_body,
        grid=(NPAD // _BN,),
        in_specs=[
            pl.BlockSpec((NC, _BN, DEGW), lambda i: (0, i, 0)),
            pl.BlockSpec((_BN, D), lambda i: (i, 0)),
        ],
        out_specs=pl.BlockSpec((_BN, D), lambda i: (i, 0)),
        out_shape=jax.ShapeDtypeStruct((NPAD, D), jnp.float32),
    )(degp, x)


def _final_body(degp_ref, aggp_ref, w_ref, b_ref, o_ref):
    deg = degp_ref[0, :, 0:1] + degp_ref[1, :, 0:1]
    dis = jnp.where(deg > 0.0, lax.rsqrt(deg), 0.0)
    acc = (aggp_ref[0] + aggp_ref[1]) * dis
    o_ref[...] = lax.dot_general(
        acc, w_ref[...], (((1,), (1,)), ((), ())),
        preferred_element_type=jnp.float32,
    ) + b_ref[...]


def _final(degp, aggp, W_w, W_b2):
    return pl.pallas_call(
        _final_body,
        grid=(NPAD // _BN,),
        in_specs=[
            pl.BlockSpec((NC, _BN, DEGW), lambda i: (0, i, 0)),
            pl.BlockSpec((NC, _BN, D), lambda i: (0, i, 0)),
            pl.BlockSpec((D, D), lambda i: (0, 0)),
            pl.BlockSpec((1, D), lambda i: (0, 0)),
        ],
        out_specs=pl.BlockSpec((_BN, D), lambda i: (i, 0)),
        out_shape=jax.ShapeDtypeStruct((NPAD, D), jnp.float32),
    )(degp, aggp, W_w, W_b2)


@jax.jit
def kernel(x, edge_index, x0, W_w, W_b):
    del x0
    row = edge_index[0]
    col = edge_index[1]
    xpad = jnp.pad(x, ((0, NPAD - N), (0, 0)))
    zeros8 = jnp.zeros((ROWS_PER_TILE, DEGW), jnp.float32)
    ones8 = jnp.ones((CHUNK, DEGW), jnp.float32)
    zerosd = jnp.zeros((ROWS_PER_TILE, D), jnp.float32)
    degp = _deg_call(col, zeros8, ones8)
    xs = _prescale(degp, xpad)
    aggp = _agg_call(row, col, xs, zerosd)
    return _final(degp, aggp, W_w, W_b.reshape(1, D))[:N]
